# baseline (speedup 1.0000x reference)
import jax
import jax.numpy as jnp
from jax.experimental import pallas as pl
from jax.experimental.pallas import tpu as pltpu

_B, _F, _T = 8, 128, 16384
_CD = 2048
_C = 1024
_NS = _CD // _C
_NT = _T // _CD


def _common(s_ref, alpha_ref, delta_ref, eps_ref, d_ref,
            w_ref, p_ref, carry_ref):
    t = pl.program_id(0)
    s = s_ref[0]
    alpha = alpha_ref[0]
    delta = delta_ref[0]
    eps = eps_ref[0]

    @pl.when(t == 0)
    def _init():
        ti = jax.lax.broadcasted_iota(jnp.int32, (_C, _C), 1)
        ji = jax.lax.broadcasted_iota(jnp.int32, (_C, _C), 0)
        d = (ti - ji).astype(jnp.float32)
        ln = jnp.log1p(-s)
        w_ref[...] = jnp.where(
            d == 0.0, s, jnp.where(d > 0.0, s * jnp.exp(d * ln), 0.0)
        )
        tt = jax.lax.broadcasted_iota(jnp.int32, (1, _C), 1).astype(jnp.float32)
        p_ref[...] = jnp.exp((tt + 1.0) * ln)
        carry_ref[...] = jnp.zeros_like(carry_ref)

    ys = []
    for h in range(_NS):
        sl = slice(h * _C, (h + 1) * _C)
        x0 = d_ref[0, :, sl]
        m = jax.lax.dot_general(
            x0, w_ref[...], (((1,), (0,)), ((), ())),
            preferred_element_type=jnp.float32,
        )
        m = m + carry_ref[...] * p_ref[...]
        carry_ref[...] = m[:, _C - 1:_C]

        madde = m + eps
        inv = jnp.exp2(alpha * -jnp.log2(madde))
        ys.append(d_ref[:, :, sl] * inv[None, :, :] + delta)
    return ys, delta


def _body_sqrt(s_ref, alpha_ref, r_ref, delta_ref, eps_ref, d_ref, o_ref,
               w_ref, p_ref, carry_ref):
    ys, delta = _common(s_ref, alpha_ref, delta_ref, eps_ref, d_ref,
                        w_ref, p_ref, carry_ref)
    sqrt_delta = delta * jax.lax.rsqrt(delta)
    for h, y in enumerate(ys):
        o_ref[:, :, h * _C:(h + 1) * _C] = y * jax.lax.rsqrt(y) - sqrt_delta


def _body_pow(s_ref, alpha_ref, r_ref, delta_ref, eps_ref, d_ref, o_ref,
              w_ref, p_ref, carry_ref):
    ys, delta = _common(s_ref, alpha_ref, delta_ref, eps_ref, d_ref,
                        w_ref, p_ref, carry_ref)
    rabs = jnp.abs(r_ref[0])
    dpow = jnp.exp2(rabs * jnp.log2(delta))
    for h, y in enumerate(ys):
        o_ref[:, :, h * _C:(h + 1) * _C] = jnp.exp2(rabs * jnp.log2(y)) - dpow


def _pcen_call(body, name, s, alpha, r, delta, eps, data):
    to_smem = lambda v: jnp.asarray(v, jnp.float32).reshape(1)
    return pl.pallas_call(
        body,
        out_shape=jax.ShapeDtypeStruct((_B, _F, _T), jnp.float32),
        grid=(_NT,),
        in_specs=[
            pl.BlockSpec(memory_space=pltpu.SMEM),
            pl.BlockSpec(memory_space=pltpu.SMEM),
            pl.BlockSpec(memory_space=pltpu.SMEM),
            pl.BlockSpec(memory_space=pltpu.SMEM),
            pl.BlockSpec(memory_space=pltpu.SMEM),
            pl.BlockSpec((_B, _F, _CD), lambda t: (0, 0, t)),
        ],
        out_specs=pl.BlockSpec((_B, _F, _CD), lambda t: (0, 0, t)),
        scratch_shapes=[
            pltpu.VMEM((_C, _C), jnp.float32),
            pltpu.VMEM((1, _C), jnp.float32),
            pltpu.VMEM((_F, 1), jnp.float32),
        ],
        compiler_params=pltpu.CompilerParams(
            dimension_semantics=("arbitrary",),
            vmem_limit_bytes=52 * 1024 * 1024,
        ),
        name=name,
    )(to_smem(s), to_smem(alpha), to_smem(r), to_smem(delta), to_smem(eps), data)


def kernel(data, alpha, r, delta, s, eps):
    return jax.lax.cond(
        jnp.abs(r) == jnp.float32(0.5),
        lambda ops: _pcen_call(_body_sqrt, "pcen_sqrt", *ops),
        lambda ops: _pcen_call(_body_pow, "pcen_pow", *ops),
        (s, alpha, r, delta, eps, data),
    )


# speedup vs baseline: 1.0756x; 1.0756x over previous
import jax
import jax.numpy as jnp
from jax.experimental import pallas as pl
from jax.experimental.pallas import tpu as pltpu

_B, _F, _T = 8, 128, 16384
_CD = 2048
_C = 512
_NS = _CD // _C
_NT = _T // _CD


def _common(s_ref, alpha_ref, delta_ref, eps_ref, d_ref,
            w_ref, p_ref, carry_ref):
    t = pl.program_id(0)
    s = s_ref[0]
    alpha = alpha_ref[0]
    delta = delta_ref[0]
    eps = eps_ref[0]

    @pl.when(t == 0)
    def _init():
        ti = jax.lax.broadcasted_iota(jnp.int32, (_C, _C), 1)
        ji = jax.lax.broadcasted_iota(jnp.int32, (_C, _C), 0)
        d = (ti - ji).astype(jnp.float32)
        ln = jnp.log1p(-s)
        w_ref[...] = jnp.where(
            d == 0.0, s, jnp.where(d > 0.0, s * jnp.exp(d * ln), 0.0)
        )
        tt = jax.lax.broadcasted_iota(jnp.int32, (1, _C), 1).astype(jnp.float32)
        p_ref[...] = jnp.exp((tt + 1.0) * ln)
        carry_ref[...] = jnp.zeros_like(carry_ref)

    ys = []
    for h in range(_NS):
        sl = slice(h * _C, (h + 1) * _C)
        x0 = d_ref[0, :, sl]
        m = jax.lax.dot_general(
            x0, w_ref[...], (((1,), (0,)), ((), ())),
            preferred_element_type=jnp.float32,
        )
        m = m + carry_ref[...] * p_ref[...]
        carry_ref[...] = m[:, _C - 1:_C]

        madde = m + eps
        inv = jnp.exp2(alpha * -jnp.log2(madde))
        ys.append(d_ref[:, :, sl] * inv[None, :, :] + delta)
    return ys, delta


def _body_sqrt(s_ref, alpha_ref, r_ref, delta_ref, eps_ref, d_ref, o_ref,
               w_ref, p_ref, carry_ref):
    ys, delta = _common(s_ref, alpha_ref, delta_ref, eps_ref, d_ref,
                        w_ref, p_ref, carry_ref)
    sqrt_delta = delta * jax.lax.rsqrt(delta)
    for h, y in enumerate(ys):
        o_ref[:, :, h * _C:(h + 1) * _C] = y * jax.lax.rsqrt(y) - sqrt_delta


def _body_pow(s_ref, alpha_ref, r_ref, delta_ref, eps_ref, d_ref, o_ref,
              w_ref, p_ref, carry_ref):
    ys, delta = _common(s_ref, alpha_ref, delta_ref, eps_ref, d_ref,
                        w_ref, p_ref, carry_ref)
    rabs = jnp.abs(r_ref[0])
    dpow = jnp.exp2(rabs * jnp.log2(delta))
    for h, y in enumerate(ys):
        o_ref[:, :, h * _C:(h + 1) * _C] = jnp.exp2(rabs * jnp.log2(y)) - dpow


def _pcen_call(body, name, s, alpha, r, delta, eps, data):
    to_smem = lambda v: jnp.asarray(v, jnp.float32).reshape(1)
    return pl.pallas_call(
        body,
        out_shape=jax.ShapeDtypeStruct((_B, _F, _T), jnp.float32),
        grid=(_NT,),
        in_specs=[
            pl.BlockSpec(memory_space=pltpu.SMEM),
            pl.BlockSpec(memory_space=pltpu.SMEM),
            pl.BlockSpec(memory_space=pltpu.SMEM),
            pl.BlockSpec(memory_space=pltpu.SMEM),
            pl.BlockSpec(memory_space=pltpu.SMEM),
            pl.BlockSpec((_B, _F, _CD), lambda t: (0, 0, t)),
        ],
        out_specs=pl.BlockSpec((_B, _F, _CD), lambda t: (0, 0, t)),
        scratch_shapes=[
            pltpu.VMEM((_C, _C), jnp.float32),
            pltpu.VMEM((1, _C), jnp.float32),
            pltpu.VMEM((_F, 1), jnp.float32),
        ],
        compiler_params=pltpu.CompilerParams(
            dimension_semantics=("arbitrary",),
            vmem_limit_bytes=52 * 1024 * 1024,
        ),
        name=name,
    )(to_smem(s), to_smem(alpha), to_smem(r), to_smem(delta), to_smem(eps), data)


def kernel(data, alpha, r, delta, s, eps):
    return jax.lax.cond(
        jnp.abs(r) == jnp.float32(0.5),
        lambda ops: _pcen_call(_body_sqrt, "pcen_sqrt", *ops),
        lambda ops: _pcen_call(_body_pow, "pcen_pow", *ops),
        (s, alpha, r, delta, eps, data),
    )
